# baseline (speedup 1.0000x reference)
import jax
import jax.numpy as jnp
from jax.experimental import pallas as pl

_B = 64
_N2 = 128
_GRID = 8
_EMB = 256
_DIM = 1024
_K = 8192


def _proj_kernel(x_ref, w_ref, b_ref, cb_ref, y_ref, cbn_ref):
    y_ref[...] = jnp.dot(x_ref[...], w_ref[...],
                         preferred_element_type=jnp.float32) + b_ref[...]
    cb = cb_ref[...]
    cbn_ref[...] = jnp.sum(cb * cb, axis=1, keepdims=True)


def _conv_kernel(xs_ref, c1_ref, c1b_ref, c2_ref, c2b_ref, z_ref):
    acc = jnp.broadcast_to(c1b_ref[...], (16 * _N2, _EMB))
    for d in range(9):
        acc = acc + jnp.dot(xs_ref[d], c1_ref[d],
                            preferred_element_type=jnp.float32)
    h = jnp.maximum(acc, 0.0)
    e = jnp.broadcast_to(c2b_ref[...], (_N2, _EMB))
    for p in range(16):
        e = e + jnp.dot(h[p * _N2:(p + 1) * _N2, :],
                        c2_ref[p * _EMB:(p + 1) * _EMB, :],
                        preferred_element_type=jnp.float32)
    z_ref[...] = e[_B:, :] - e[:_B, :]


def _vq_kernel(z_ref, cb_ref, cbn_ref, rv_ref, wout_ref, bout_ref,
               out_ref, perp_ref):
    z = z_ref[...]
    cb = cb_ref[...]
    cross = jax.lax.dot_general(z, cb, (((1,), (1,)), ((), ())),
                                preferred_element_type=jnp.float32)
    s = cbn_ref[...] - 2.0 * cross
    smin = jnp.min(s, axis=1, keepdims=True)
    iota = jax.lax.broadcasted_iota(jnp.int32, (_B, _K), 1)
    idx = jnp.min(jnp.where(s <= smin, iota, _K), axis=1, keepdims=True)
    onehot = (iota == idx).astype(jnp.float32)
    hard = jnp.dot(onehot, cb, preferred_element_type=jnp.float32)

    dz = z - hard
    nq = jnp.sqrt(jnp.sum(dz * dz, axis=1, keepdims=True))
    rv = rv_ref[...]
    nr = jnp.sqrt(jnp.sum(rv * rv, axis=1, keepdims=True))
    quantized = z + (nq / (nr + 1e-12)) * rv

    ieq = (idx == jnp.transpose(idx)).astype(jnp.float32)
    counts = jnp.sum(ieq, axis=1, keepdims=True)
    lp = jnp.log(counts * (1.0 / _B) + 1e-10)
    perp_ref[...] = jnp.broadcast_to(jnp.exp(-jnp.sum(lp) * (1.0 / _B)), (1, 1))

    out_ref[...] = (jnp.dot(quantized, wout_ref[...],
                            preferred_element_type=jnp.float32) + bout_ref[...])


def kernel(input_data_first, input_data_last, codebooks, Win, b_in, Wout,
           b_out, c1w, c1b, c2w, c2b):
    x_cat = jnp.concatenate([input_data_first, input_data_last],
                            axis=0).reshape(_N2 * _GRID * _GRID, _DIM)
    y, cbn = pl.pallas_call(
        _proj_kernel,
        grid=(8,),
        in_specs=[
            pl.BlockSpec((1024, _DIM), lambda i: (i, 0)),
            pl.BlockSpec((_DIM, _EMB), lambda i: (0, 0)),
            pl.BlockSpec((1, _EMB), lambda i: (0, 0)),
            pl.BlockSpec((1024, _EMB), lambda i: (i, 0)),
        ],
        out_specs=[
            pl.BlockSpec((1024, _EMB), lambda i: (i, 0)),
            pl.BlockSpec((1024, 1), lambda i: (i, 0)),
        ],
        out_shape=[
            jax.ShapeDtypeStruct((_N2 * _GRID * _GRID, _EMB), jnp.float32),
            jax.ShapeDtypeStruct((_K, 1), jnp.float32),
        ],
    )(x_cat, Win, b_in.reshape(1, _EMB), codebooks)

    ypad = jnp.pad(y.reshape(_N2, _GRID, _GRID, _EMB),
                   ((0, 0), (1, 1), (1, 1), (0, 0)))
    xs = jnp.stack([
        ypad[:, di:di + 8:2, dj:dj + 8:2, :]
        .transpose(1, 2, 0, 3).reshape(16 * _N2, _EMB)
        for di in range(3) for dj in range(3)])

    c1r = jnp.transpose(c1w, (2, 3, 1, 0)).reshape(9, _EMB, _EMB)
    c2r = jnp.transpose(c2w, (2, 3, 1, 0)).reshape(16 * _EMB, _EMB)
    rv = jax.random.normal(jax.random.key(42), (_B, _EMB), dtype=jnp.float32)

    z = pl.pallas_call(
        _conv_kernel,
        out_shape=jax.ShapeDtypeStruct((_B, _EMB), jnp.float32),
    )(xs, c1r, c1b.reshape(1, _EMB), c2r, c2b.reshape(1, _EMB))

    out, perp = pl.pallas_call(
        _vq_kernel,
        out_shape=[
            jax.ShapeDtypeStruct((_B, _DIM), jnp.float32),
            jax.ShapeDtypeStruct((1, 1), jnp.float32),
        ],
    )(z, codebooks, cbn.reshape(1, _K), rv, Wout, b_out.reshape(1, _DIM))
    return out.reshape(_B, 1, _DIM), perp.reshape(())


# speedup vs baseline: 2.1022x; 2.1022x over previous
import jax
import jax.numpy as jnp
from jax.experimental import pallas as pl

_B = 64
_GRID = 8
_EMB = 256
_DIM = 1024
_K = 8192
_CH = 16


def _enc_kernel(x_ref, w_ref, b_ref, c1_ref, c1b_ref, c2_ref, c2b_ref,
                e_ref):
    y = jnp.dot(x_ref[...], w_ref[...],
                preferred_element_type=jnp.float32) + b_ref[...]
    y4 = y.reshape(_CH, _GRID, _GRID, _EMB)
    yp = jnp.pad(y4, ((0, 0), (1, 1), (1, 1), (0, 0)))
    y6 = yp.reshape(_CH, 5, 2, 5, 2, _EMB)
    acc = jnp.broadcast_to(c1b_ref[...], (_CH * 16, _EMB))
    for di in range(3):
        fi, pi = di // 2, di % 2
        for dj in range(3):
            fj, pj = dj // 2, dj % 2
            xs = y6[:, fi:fi + 4, pi, fj:fj + 4, pj, :].reshape(
                _CH * 16, _EMB)
            acc = acc + jnp.dot(xs, c1_ref[di * 3 + dj],
                                preferred_element_type=jnp.float32)
    h = jnp.maximum(acc, 0.0).reshape(_CH, 16 * _EMB)
    e_ref[...] = jnp.dot(h, c2_ref[...],
                         preferred_element_type=jnp.float32) + c2b_ref[...]


def _vq_kernel(e1_ref, e2_ref, cb_ref, rv_ref, wout_ref, bout_ref,
               ones_ref, out_ref, perp_ref):
    z = e2_ref[...] - e1_ref[...]
    cb = cb_ref[...]
    cbn = jax.lax.dot_general(ones_ref[...], cb * cb, (((1,), (1,)), ((), ())),
                              preferred_element_type=jnp.float32)
    cross = jax.lax.dot_general(z, cb, (((1,), (1,)), ((), ())),
                                preferred_element_type=jnp.float32)
    s = cbn - 2.0 * cross
    smin = jnp.min(s, axis=1, keepdims=True)
    iota = jax.lax.broadcasted_iota(jnp.int32, (_B, _K), 1)
    idx = jnp.min(jnp.where(s <= smin, iota, _K), axis=1, keepdims=True)
    onehot = (iota == idx).astype(jnp.float32)
    hard = jnp.dot(onehot, cb, preferred_element_type=jnp.float32)

    dz = z - hard
    nq = jnp.sqrt(jnp.sum(dz * dz, axis=1, keepdims=True))
    rv = rv_ref[...]
    nr = jnp.sqrt(jnp.sum(rv * rv, axis=1, keepdims=True))
    quantized = z + (nq / (nr + 1e-12)) * rv

    ieq = (idx == jnp.transpose(idx)).astype(jnp.float32)
    counts = jnp.sum(ieq, axis=1, keepdims=True)
    lp = jnp.log(counts * (1.0 / _B) + 1e-10)
    perp_ref[...] = jnp.broadcast_to(jnp.exp(-jnp.sum(lp) * (1.0 / _B)), (1, 1))

    out_ref[...] = (jnp.dot(quantized, wout_ref[...],
                            preferred_element_type=jnp.float32) + bout_ref[...])


def kernel(input_data_first, input_data_last, codebooks, Win, b_in, Wout,
           b_out, c1w, c1b, c2w, c2b):
    c1r = jnp.transpose(c1w, (2, 3, 1, 0)).reshape(9, _EMB, _EMB)
    c2r = jnp.transpose(c2w, (2, 3, 1, 0)).reshape(16 * _EMB, _EMB)
    rv = jax.random.normal(jax.random.key(42), (_B, _EMB), dtype=jnp.float32)
    b_in2 = b_in.reshape(1, _EMB)
    c1b2 = c1b.reshape(1, _EMB)
    c2b2 = c2b.reshape(1, _EMB)

    enc = pl.pallas_call(
        _enc_kernel,
        grid=(_B // _CH,),
        in_specs=[
            pl.BlockSpec((_CH * 64, _DIM), lambda i: (i, 0)),
            pl.BlockSpec((_DIM, _EMB), lambda i: (0, 0)),
            pl.BlockSpec((1, _EMB), lambda i: (0, 0)),
            pl.BlockSpec((9, _EMB, _EMB), lambda i: (0, 0, 0)),
            pl.BlockSpec((1, _EMB), lambda i: (0, 0)),
            pl.BlockSpec((16 * _EMB, _EMB), lambda i: (0, 0)),
            pl.BlockSpec((1, _EMB), lambda i: (0, 0)),
        ],
        out_specs=pl.BlockSpec((_CH, _EMB), lambda i: (i, 0)),
        out_shape=jax.ShapeDtypeStruct((_B, _EMB), jnp.float32),
    )
    e1 = enc(input_data_first.reshape(_B * 64, _DIM), Win, b_in2, c1r, c1b2,
             c2r, c2b2)
    e2 = enc(input_data_last.reshape(_B * 64, _DIM), Win, b_in2, c1r, c1b2,
             c2r, c2b2)

    out, perp = pl.pallas_call(
        _vq_kernel,
        out_shape=[
            jax.ShapeDtypeStruct((_B, _DIM), jnp.float32),
            jax.ShapeDtypeStruct((1, 1), jnp.float32),
        ],
    )(e1, e2, codebooks, rv, Wout, b_out.reshape(1, _DIM),
      jnp.ones((1, _EMB), dtype=jnp.float32))
    return out.reshape(_B, 1, _DIM), perp.reshape(())
